# baseline (speedup 1.0000x reference)
import functools

import jax
import jax.numpy as jnp
from jax import lax
from jax.experimental import pallas as pl
from jax.experimental.pallas import tpu as pltpu
from jax.experimental.pallas import tpu_sc as plsc

OUT_SIZE = 64
PAD_W = 128
BATCH = 16384
HIST = 50
HIST_P = 56
ROWS = BATCH * HIST_P

NC, NS = 2, 16
NW = NC * NS
ROWS_W = ROWS // NW
STREAM = 128
NCHUNK = ROWS_W // STREAM
NBUF = 4


def _gather_body(idx_hbm, table_hbm, out_hbm, idx_v, rows_v, *sems):
    gsems, ssems = sems[:NBUF], sems[NBUF:]
    wid = lax.axis_index("s") * NC + lax.axis_index("c")
    pltpu.sync_copy(idx_hbm.at[wid], idx_v)
    out_base = wid * ROWS_W

    def issue_gather(j, b):
        pltpu.async_copy(table_hbm.at[idx_v.at[j]], rows_v.at[b], gsems[b])

    def wait_gather(j, b):
        pltpu.make_async_copy(
            table_hbm.at[idx_v.at[j]], rows_v.at[b], gsems[b]).wait()

    def store_descr(j, b):
        return (rows_v.at[b],
                out_hbm.at[pl.ds(out_base + j * STREAM, STREAM)], ssems[b])

    for b in range(NBUF - 1):
        issue_gather(b, b)

    def body(g, carry):
        for b in range(NBUF):
            j = g * NBUF + b
            bp = (b + NBUF - 1) % NBUF
            wait_gather(j, b)
            pltpu.async_copy(*store_descr(j, b))

            @pl.when(j >= 1)
            def _():
                pltpu.make_async_copy(*store_descr(j - 1, bp)).wait()

            @pl.when(j + NBUF - 1 < NCHUNK)
            def _():
                issue_gather(j + NBUF - 1, bp)
        return carry

    lax.fori_loop(0, NCHUNK // NBUF, body, 0)
    pltpu.make_async_copy(*store_descr(NCHUNK - 1, (NCHUNK - 1) % NBUF)).wait()


@functools.partial(jax.jit, static_argnums=())
def _run(idx, table):
    k = pl.kernel(
        _gather_body,
        out_type=jax.ShapeDtypeStruct((ROWS, OUT_SIZE), jnp.float32),
        mesh=plsc.VectorSubcoreMesh(core_axis_name="c", subcore_axis_name="s"),
        scratch_types=[
            pltpu.VMEM((NCHUNK, STREAM), jnp.int32),
            pltpu.VMEM((NBUF, STREAM, OUT_SIZE), jnp.float32),
        ] + [pltpu.SemaphoreType.DMA] * (2 * NBUF),
        compiler_params=pltpu.CompilerParams(use_tc_tiling_on_sc=False),
    )
    return k(idx, table)


def kernel(inputs, embeddings):
    idx = jnp.pad(inputs.astype(jnp.int32), ((0, 0), (0, HIST_P - HIST)))
    idx = (idx * 2).reshape(NW, NCHUNK, STREAM)
    tbl = jnp.pad(embeddings, ((0, 0), (0, PAD_W - OUT_SIZE)))
    tbl = tbl.reshape(2 * 1000000, OUT_SIZE)
    out = _run(idx, tbl)
    return out.reshape(BATCH, HIST_P, OUT_SIZE)[:, :HIST, :]
